# baseline (speedup 1.0000x reference)
import functools

import jax
import jax.numpy as jnp
from jax import lax
from jax.experimental import pallas as pl
from jax.experimental.pallas import tpu as pltpu

NE = 1024
ED = 64
BLK = 2048
CC = 0.25


def _vq_body(ntok, x_ref, emb_ref, embt_ref,
             outq_ref, enc_ref, loss_ref, perp_ref, cnt_ref):
    b = pl.program_id(0)
    j = pl.program_id(1)
    first = jnp.logical_and(b == 0, j == 0)
    last = jnp.logical_and(b == pl.num_programs(0) - 1,
                           j == pl.num_programs(1) - 1)

    @pl.when(first)
    def _():
        loss_ref[...] = jnp.zeros_like(loss_ref)
        cnt_ref[...] = jnp.zeros_like(cnt_ref)

    xv = x_ref[0].reshape(x_ref.shape[1], -1)
    xb = xv.T
    embt = embt_ref[...]
    scores = jnp.dot(xb, embt, preferred_element_type=jnp.float32)
    x2 = jnp.sum(xb * xb, axis=1, keepdims=True)
    e2 = jnp.sum(embt * embt, axis=0, keepdims=True)
    d = (x2 + e2) - 2.0 * scores
    m = jnp.min(d, axis=1, keepdims=True)
    iot = lax.broadcasted_iota(jnp.int32, (1, NE), 1).astype(jnp.float32)
    idx = jnp.min(jnp.where(d == m, iot, float(NE)), axis=1, keepdims=True)
    enc = (iot == idx).astype(jnp.float32)
    enc_ref[...] = enc
    q = jnp.dot(enc, emb_ref[...], preferred_element_type=jnp.float32)
    outq_ref[0] = q.T
    loss_ref[...] += jnp.sum(m, keepdims=True).reshape(1, 1)
    cnt_ref[...] += jnp.sum(enc, axis=0, keepdims=True)

    @pl.when(last)
    def _():
        p = cnt_ref[...] * (1.0 / ntok)
        ent = jnp.sum(p * jnp.log(p + 1e-10), axis=1, keepdims=True)
        perp_ref[...] = jnp.exp(-ent)
        loss_ref[...] = loss_ref[...] * ((1.0 + CC) / (ntok * ED))


def kernel(inputs, embedding):
    B, C, D, H, W = inputs.shape
    S = D * H * W
    ntok = B * S
    nj = S // BLK
    xr = inputs.reshape(B, C, S // W, W)
    embt = embedding.T

    out_shapes = (
        jax.ShapeDtypeStruct((B, C, S), jnp.float32),
        jax.ShapeDtypeStruct((ntok, NE), jnp.float32),
        jax.ShapeDtypeStruct((1, 1), jnp.float32),
        jax.ShapeDtypeStruct((1, 1), jnp.float32),
    )
    outq, enc, loss, perp = pl.pallas_call(
        functools.partial(_vq_body, ntok),
        grid=(B, nj),
        in_specs=[
            pl.BlockSpec((1, C, BLK // W, W), lambda b, j: (b, 0, j, 0)),
            pl.BlockSpec((NE, ED), lambda b, j: (0, 0)),
            pl.BlockSpec((ED, NE), lambda b, j: (0, 0)),
        ],
        out_specs=(
            pl.BlockSpec((1, C, BLK), lambda b, j: (b, 0, j)),
            pl.BlockSpec((BLK, NE), lambda b, j: (b * nj + j, 0)),
            pl.BlockSpec((1, 1), lambda b, j: (0, 0)),
            pl.BlockSpec((1, 1), lambda b, j: (0, 0)),
        ),
        out_shape=out_shapes,
        scratch_shapes=[pltpu.VMEM((1, NE), jnp.float32)],
    )(xr, embedding, embt)

    out_q = outq.reshape(B, C, D, H, W)
    return (loss[0, 0], out_q, perp[0, 0], enc)


# speedup vs baseline: 1.6223x; 1.6223x over previous
import functools

import jax
import jax.numpy as jnp
from jax import lax
from jax.experimental import pallas as pl
from jax.experimental.pallas import tpu as pltpu

NE = 1024
ED = 64
BLK = 4096
CC = 0.25


def _vq_body(ntok, x_ref, emb_ref, embt_ref,
             outq_ref, enc_ref, loss_ref, perp_ref, cnt_ref):
    j = pl.program_id(0)
    first = j == 0
    last = j == pl.num_programs(0) - 1

    @pl.when(first)
    def _():
        loss_ref[...] = jnp.zeros_like(loss_ref)
        cnt_ref[...] = jnp.zeros_like(cnt_ref)

    xb = x_ref[...]
    embt = embt_ref[...]
    scores = jnp.dot(xb, embt, preferred_element_type=jnp.float32)
    x2 = jnp.sum(xb * xb, axis=1, keepdims=True)
    e2 = jnp.sum(embt * embt, axis=0, keepdims=True)
    d = (x2 + e2) - 2.0 * scores
    m = jnp.min(d, axis=1, keepdims=True)
    iot = lax.broadcasted_iota(jnp.int32, (1, NE), 1).astype(jnp.float32)
    idx = jnp.min(jnp.where(d == m, iot, float(NE)), axis=1, keepdims=True)
    enc = (iot == idx).astype(jnp.float32)
    enc_ref[...] = enc
    outq_ref[...] = jnp.dot(enc, emb_ref[...], preferred_element_type=jnp.float32)
    loss_ref[...] += jnp.sum(m, keepdims=True).reshape(1, 1)
    cnt_ref[...] += jnp.sum(enc, axis=0, keepdims=True)

    @pl.when(last)
    def _():
        p = cnt_ref[...] * (1.0 / ntok)
        ent = jnp.sum(p * jnp.log(p + 1e-10), axis=1, keepdims=True)
        perp_ref[...] = jnp.exp(-ent)
        loss_ref[...] = loss_ref[...] * ((1.0 + CC) / (ntok * ED))


def kernel(inputs, embedding):
    B, C, D, H, W = inputs.shape
    ntok = B * D * H * W
    nj = ntok // BLK
    flat = jnp.transpose(inputs, (0, 2, 3, 4, 1)).reshape(ntok, C)
    embt = embedding.T

    out_shapes = (
        jax.ShapeDtypeStruct((ntok, C), jnp.float32),
        jax.ShapeDtypeStruct((ntok, NE), jnp.float32),
        jax.ShapeDtypeStruct((1, 1), jnp.float32),
        jax.ShapeDtypeStruct((1, 1), jnp.float32),
    )
    flat_q, enc, loss, perp = pl.pallas_call(
        functools.partial(_vq_body, ntok),
        grid=(nj,),
        in_specs=[
            pl.BlockSpec((BLK, C), lambda j: (j, 0)),
            pl.BlockSpec((NE, ED), lambda j: (0, 0)),
            pl.BlockSpec((ED, NE), lambda j: (0, 0)),
        ],
        out_specs=(
            pl.BlockSpec((BLK, C), lambda j: (j, 0)),
            pl.BlockSpec((BLK, NE), lambda j: (j, 0)),
            pl.BlockSpec((1, 1), lambda j: (0, 0)),
            pl.BlockSpec((1, 1), lambda j: (0, 0)),
        ),
        out_shape=out_shapes,
        scratch_shapes=[pltpu.VMEM((1, NE), jnp.float32)],
    )(flat, embedding, embt)

    out_q = jnp.transpose(flat_q.reshape(B, D, H, W, C), (0, 4, 1, 2, 3))
    return (loss[0, 0], out_q, perp[0, 0], enc)
